# Initial kernel scaffold; baseline (speedup 1.0000x reference)
import jax
import jax.numpy as jnp
from jax.experimental import pallas as pl


def kernel(pred_logits, pred_boxes, orig_target_sizes):
    raise NotImplementedError("write your pallas kernel here")


# speedup vs baseline: 5.3297x; 5.3297x over previous
import jax
import jax.numpy as jnp
from jax.experimental import pallas as pl
from jax.experimental.pallas import tpu as pltpu

B, N, C = 16, 20000, 80
K = 300
LANES = 128
ROWS = (N * C) // LANES
GROUPS = 98
RPAD = GROUPS * LANES
NEG = -3.0e38


def _post_kernel(flat_ref, boxes_ref, scale_ref,
                 lab_ref, lo_ref, hi_ref, sc_ref,
                 data, gmax):
    data[ROWS:RPAD, :] = jnp.full((RPAD - ROWS, LANES), NEG, jnp.float32)
    data[0:ROWS, :] = flat_ref[0]
    gmax[...] = jnp.max(data[...].reshape(GROUPS, LANES, LANES), axis=1)

    s2 = scale_ref[0, 0, :]

    giota = jax.lax.broadcasted_iota(jnp.int32, (GROUPS, LANES), 0)
    bkey = (jax.lax.broadcasted_iota(jnp.int32, (LANES, LANES), 0) * LANES
            + jax.lax.broadcasted_iota(jnp.int32, (LANES, LANES), 1))
    laneiota = jax.lax.broadcasted_iota(jnp.int32, (1, LANES), 1)
    BIG = jnp.int32(2**30)

    def body(k, _):
        gm = gmax[...]
        m = jnp.max(gm)
        g = jnp.min(jnp.where(gm == m, giota, BIG))
        gs = g * LANES
        bloc = data[pl.ds(gs, LANES), :]
        k2 = jnp.min(jnp.where(bloc == m, bkey, BIG))
        r_in = k2 // LANES
        l = k2 - r_in * LANES
        r = gs + r_in
        flat_idx = r * LANES + l

        row = data[pl.ds(r, 1), :]
        data[pl.ds(r, 1), :] = jnp.where(laneiota == l, NEG, row)
        bloc_new = jnp.where(bkey == k2, NEG, bloc)
        gmax[pl.ds(g, 1), :] = jnp.max(bloc_new, axis=0)[None, :]

        q = flat_idx // C
        lab_ref[0, pl.ds(k, 1), 0] = jnp.reshape(flat_idx - q * C, (1,))
        sc_ref[0, pl.ds(k, 1), 0] = jnp.reshape(m, (1,))

        brow = boxes_ref[0, pl.ds(q, 1), :]
        c2 = brow[:, 0:2]
        half = 0.5 * brow[:, 2:4]
        lo_ref[0, pl.ds(k, 1), :] = (c2 - half) * s2
        hi_ref[0, pl.ds(k, 1), :] = (c2 + half) * s2
        return 0

    jax.lax.fori_loop(0, K, body, 0)
    sc_ref[0, :, :] = jax.nn.sigmoid(sc_ref[0, :, :])


def kernel(pred_logits, pred_boxes, orig_target_sizes):
    flat = pred_logits.reshape(B, ROWS, LANES)
    scale = orig_target_sizes.astype(jnp.float32).reshape(B, 1, 2)

    labels, lo, hi, scores = pl.pallas_call(
        _post_kernel,
        grid=(B,),
        in_specs=[
            pl.BlockSpec((1, ROWS, LANES), lambda b: (b, 0, 0)),
            pl.BlockSpec((1, N, 4), lambda b: (b, 0, 0)),
            pl.BlockSpec((1, 1, 2), lambda b: (b, 0, 0)),
        ],
        out_specs=[
            pl.BlockSpec((1, K, 1), lambda b: (b, 0, 0)),
            pl.BlockSpec((1, K, 2), lambda b: (b, 0, 0)),
            pl.BlockSpec((1, K, 2), lambda b: (b, 0, 0)),
            pl.BlockSpec((1, K, 1), lambda b: (b, 0, 0)),
        ],
        out_shape=[
            jax.ShapeDtypeStruct((B, K, 1), jnp.int32),
            jax.ShapeDtypeStruct((B, K, 2), jnp.float32),
            jax.ShapeDtypeStruct((B, K, 2), jnp.float32),
            jax.ShapeDtypeStruct((B, K, 1), jnp.float32),
        ],
        scratch_shapes=[
            pltpu.VMEM((RPAD, LANES), jnp.float32),
            pltpu.VMEM((GROUPS, LANES), jnp.float32),
        ],
    )(flat, pred_boxes, scale)

    boxes = jnp.concatenate([lo, hi], axis=-1)
    return labels[:, :, 0], boxes, scores[:, :, 0]
